# baseline (speedup 1.0000x reference)
import functools

import jax
import jax.numpy as jnp
from jax import lax
from jax.experimental import pallas as pl
from jax.experimental.pallas import tpu as pltpu
from jax.experimental.pallas import tpu_sc as plsc

N_NODES = 10000
N_EDGES = 320000
DIM = 128
N_GRAPHS = 64
N_LAYERS = 3

NC = 2
NS = 16
NW = NC * NS

CHUNK = 128
E_PAD = 327680
N_CHUNKS = E_PAD // CHUNK
CPW = N_CHUNKS // NW
BLK = 16
NBLK = CPW // BLK
N_PAD = 10112
ROWS_PER_TILE = N_PAD // NS


def _sc_aggregate(h, src2d, dst2d, zeros_tile):
    mesh = plsc.VectorSubcoreMesh(
        core_axis_name="c", subcore_axis_name="s", num_cores=NC, num_subcores=NS
    )

    @functools.partial(
        pl.kernel,
        out_type=jax.ShapeDtypeStruct((NC, N_PAD, DIM), jnp.float32),
        mesh=mesh,
        scratch_types=[
            pltpu.VMEM((2 * BLK, CHUNK), jnp.int32),
            pltpu.VMEM((2 * BLK, CHUNK), jnp.int32),
            pltpu.VMEM((2 * CHUNK, DIM), jnp.float32),
            pltpu.VMEM_SHARED((N_PAD, DIM), jnp.float32),
            pltpu.SemaphoreType.DMA,
            pltpu.SemaphoreType.DMA,
            pltpu.SemaphoreType.DMA,
            pltpu.SemaphoreType.DMA,
            pltpu.SemaphoreType.DMA,
        ],
    )
    def body(h_hbm, src_hbm, dst_hbm, zero_hbm, out_hbm, sidx, didx, rows,
             acc_sh, gsem0, gsem1, ssem0, ssem1, isem):
        c = lax.axis_index("c")
        s = lax.axis_index("s")
        w = s * NC + c
        base_chunk = w * CPW
        gsem = (gsem0, gsem1)
        ssem = (ssem0, ssem1)

        def rows_at(b):
            return rows.at[pl.ds(b * CHUNK, CHUNK)]

        def wait_g(b):
            pltpu.make_async_copy(zero_hbm.at[pl.ds(0, CHUNK)], rows_at(b),
                                  gsem[b]).wait()

        def wait_s(b):
            pltpu.make_async_copy(rows_at(b), acc_sh.at[pl.ds(0, CHUNK)],
                                  ssem[b]).wait()

        def wait_i():
            pltpu.make_async_copy(src_hbm.at[pl.ds(0, BLK)],
                                  sidx.at[pl.ds(0, BLK)], isem).wait()

        def fire_gather(b, idx_row):
            pltpu.async_copy(h_hbm.at[sidx.at[idx_row]], rows_at(b), gsem[b])

        def fire_scatter(b, idx_row):
            pltpu.async_copy(rows_at(b), acc_sh.at[didx.at[idx_row]],
                             ssem[b], add=True)

        pltpu.sync_copy(zero_hbm,
                        acc_sh.at[pl.ds(s * ROWS_PER_TILE, ROWS_PER_TILE)])
        plsc.subcore_barrier()

        pltpu.sync_copy(src_hbm.at[pl.ds(base_chunk, BLK)], sidx.at[pl.ds(0, BLK)])
        pltpu.sync_copy(dst_hbm.at[pl.ds(base_chunk, BLK)], didx.at[pl.ds(0, BLK)])

        def block(t, carry):
            cur = (t % 2) * BLK
            prev = ((t + 1) % 2) * BLK

            @pl.when(t > 0)
            def _wait_idx():
                wait_i()
                wait_i()

            for p in range(BLK):
                b = p % 2
                bo = 1 - b
                if p >= 2:
                    wait_s(b)
                    fire_gather(b, cur + p)
                    wait_g(bo)
                    fire_scatter(bo, cur + p - 1)
                elif p == 1:
                    @pl.when(t > 0)
                    def _w1():
                        wait_s(b)
                    fire_gather(b, cur + 1)
                    wait_g(bo)
                    fire_scatter(bo, cur)
                else:
                    @pl.when(t > 0)
                    def _p0():
                        wait_s(b)
                        fire_gather(b, cur)
                        wait_g(bo)
                        fire_scatter(bo, prev + BLK - 1)

                    @pl.when(t == 0)
                    def _p0_first():
                        fire_gather(0, 0)
                if p == 2:
                    @pl.when(t < NBLK - 1)
                    def _prefetch():
                        off = base_chunk + (t + 1) * BLK
                        pltpu.async_copy(src_hbm.at[pl.ds(off, BLK)],
                                         sidx.at[pl.ds(prev, BLK)], isem)
                        pltpu.async_copy(dst_hbm.at[pl.ds(off, BLK)],
                                         didx.at[pl.ds(prev, BLK)], isem)
            return carry

        lax.fori_loop(0, NBLK, block, 0)

        last = (CPW - 1) % 2
        wait_g(last)
        fire_scatter(last, ((NBLK - 1) % 2) * BLK + BLK - 1)
        wait_s(0)
        wait_s(1)

        plsc.subcore_barrier()

        pltpu.sync_copy(
            acc_sh.at[pl.ds(s * ROWS_PER_TILE, ROWS_PER_TILE)],
            out_hbm.at[c, pl.ds(s * ROWS_PER_TILE, ROWS_PER_TILE)],
        )

    return body(h, src2d, dst2d, zeros_tile)


def _tc_layer(h, acc, epsp1, W1, b1, g1, be1, W2, b2, go, bo, batch2d):
    def body(eps_ref, h_ref, a_ref, w1_ref, b1_ref, g1_ref, be1_ref,
             w2_ref, b2_ref, go_ref, bo_ref, batch_ref, out_ref, pool_ref):
        ep = eps_ref[0]
        z = ep * h_ref[...] + a_ref[0, :N_NODES] + a_ref[1, :N_NODES]
        z1 = lax.dot_general(z, w1_ref[...], (((1,), (0,)), ((), ())),
                             preferred_element_type=jnp.float32) + b1_ref[...]
        mu1 = jnp.mean(z1, axis=0, keepdims=True)
        d1 = z1 - mu1
        var1 = jnp.mean(d1 * d1, axis=0, keepdims=True)
        z1n = jnp.maximum(
            g1_ref[...] * d1 * lax.rsqrt(var1 + 1e-5) + be1_ref[...], 0.0)
        z2 = lax.dot_general(z1n, w2_ref[...], (((1,), (0,)), ((), ())),
                             preferred_element_type=jnp.float32) + b2_ref[...]
        mu2 = jnp.mean(z2, axis=0, keepdims=True)
        d2 = z2 - mu2
        var2 = jnp.mean(d2 * d2, axis=0, keepdims=True)
        h_out = jnp.maximum(
            go_ref[...] * d2 * lax.rsqrt(var2 + 1e-5) + bo_ref[...], 0.0)
        out_ref[...] = h_out

        gids = lax.broadcasted_iota(jnp.int32, (N_GRAPHS, N_NODES), 0)
        onehot = (gids == batch_ref[...]).astype(jnp.float32)
        pool_ref[...] = lax.dot_general(
            onehot, h_out, (((1,), (0,)), ((), ())),
            preferred_element_type=jnp.float32)

    return pl.pallas_call(
        body,
        out_shape=(
            jax.ShapeDtypeStruct((N_NODES, DIM), jnp.float32),
            jax.ShapeDtypeStruct((N_GRAPHS, DIM), jnp.float32),
        ),
        in_specs=[pl.BlockSpec(memory_space=pltpu.SMEM)]
        + [pl.BlockSpec(memory_space=pltpu.VMEM)] * 11,
    )(epsp1, h, acc, W1, b1, g1, be1, W2, b2, go, bo, batch2d)


def kernel(x, edge_index, batch, eps, W1, b1, g1, be1, W2, b2, go, bo):
    n_fill = E_PAD - N_EDGES
    src_pad = jnp.concatenate(
        [edge_index[0], jnp.zeros((n_fill,), jnp.int32)])
    dst_pad = jnp.concatenate(
        [edge_index[1],
         N_NODES + (jnp.arange(n_fill, dtype=jnp.int32) % (N_PAD - N_NODES))])
    src2d = src_pad.reshape(N_CHUNKS, CHUNK)
    dst2d = dst_pad.reshape(N_CHUNKS, CHUNK)
    zeros_tile = jnp.zeros((ROWS_PER_TILE, DIM), jnp.float32)
    batch2d = batch.reshape(1, N_NODES)
    epsp1 = (1.0 + eps).astype(jnp.float32)

    h = x
    reps = []
    pooled = []
    for i in range(N_LAYERS):
        acc = _sc_aggregate(h, src2d, dst2d, zeros_tile)
        h, p = _tc_layer(
            h, acc, epsp1[i].reshape(1),
            W1[i], b1[i].reshape(1, DIM), g1[i].reshape(1, DIM),
            be1[i].reshape(1, DIM), W2[i], b2[i].reshape(1, DIM),
            go[i].reshape(1, DIM), bo[i].reshape(1, DIM), batch2d)
        reps.append(h)
        pooled.append(p)

    node_rep = jnp.concatenate(reps, axis=1)
    graph_rep = jnp.concatenate(pooled, axis=1)
    return (graph_rep, node_rep)
